# baseline (speedup 1.0000x reference)
import functools

import jax
import jax.numpy as jnp
from jax.experimental import pallas as pl
from jax.experimental.pallas import tpu as pltpu
from jax.experimental.pallas import tpu_sc as plsc

B = 4
N = 16384
NPT = 1024
RAD2 = (0.25, 1.0)
NS = (16, 32)


def _fps_body(x_ref, y_ref, z_ref, ox_ref, oy_ref, oz_ref, d_ref):
    x = x_ref[...]
    y = y_ref[...]
    z = z_ref[...]
    lin = (jax.lax.broadcasted_iota(jnp.int32, (1, 128, 128), 1) * 128
           + jax.lax.broadcasted_iota(jnp.int32, (1, 128, 128), 2))
    ii = jax.lax.broadcasted_iota(jnp.int32, (B, NPT), 1)
    d_ref[...] = jnp.full((B, 128, 128), 1e10, jnp.float32)

    def step(i, far):
        sel = lin == far
        cx = jnp.sum(jnp.where(sel, x, 0.0), axis=(1, 2), keepdims=True)
        cy = jnp.sum(jnp.where(sel, y, 0.0), axis=(1, 2), keepdims=True)
        cz = jnp.sum(jnp.where(sel, z, 0.0), axis=(1, 2), keepdims=True)
        hit = ii == i
        ox_ref[...] = jnp.where(hit, cx[:, :, 0], ox_ref[...])
        oy_ref[...] = jnp.where(hit, cy[:, :, 0], oy_ref[...])
        oz_ref[...] = jnp.where(hit, cz[:, :, 0], oz_ref[...])
        d = (x - cx) ** 2 + (y - cy) ** 2 + (z - cz) ** 2
        dn = jnp.minimum(d_ref[...], d)
        d_ref[...] = dn
        m = jnp.max(dn, axis=(1, 2), keepdims=True)
        return jnp.min(jnp.where(dn == m, lin, jnp.int32(N)),
                       axis=(1, 2), keepdims=True)

    jax.lax.fori_loop(0, NPT, step, jnp.zeros((B, 1, 1), jnp.int32))


def _fps(xs, ys, zs):
    out = jax.ShapeDtypeStruct((B, NPT), jnp.float32)
    return pl.pallas_call(
        _fps_body,
        out_shape=[out, out, out],
        scratch_shapes=[pltpu.VMEM((B, 128, 128), jnp.float32)],
    )(xs, ys, zs)


_CB = 256
_JC = 2048
_NJ = N // _JC
_KB = 8


def _cumsum_lanes(m):
    x = m
    k = 1
    while k < _JC:
        x = x + jnp.concatenate(
            [jnp.zeros((_CB, k), jnp.int32), x[:, :-k]], axis=1)
        k *= 2
    return x


def _select_body(x_ref, c_ref, o1_ref, o2_ref, car1, car2):
    j = pl.program_id(2)

    @pl.when(j == 0)
    def _():
        car1[...] = jnp.zeros((_CB, 1), jnp.int32)
        car2[...] = jnp.zeros((_CB, 1), jnp.int32)
        o1_ref[...] = jnp.zeros((1, _CB, NS[0]), jnp.int32)
        o2_ref[...] = jnp.zeros((1, _CB, NS[1]), jnp.int32)

    jg = (jax.lax.broadcasted_iota(jnp.int32, (1, _JC), 1)
          + j * _JC)

    def scale_work(d2, mask, ns, o_ref, car, newcar):
        pos = car[...] + _cumsum_lanes(mask.astype(jnp.int32))
        for b in range(ns // _KB):
            def band(b=b):
                cols = [jnp.sum(jnp.where(jnp.logical_and(pos == k + 1, mask),
                                          jg, 0),
                                axis=1, keepdims=True, dtype=jnp.int32)
                        for k in range(b * _KB, (b + 1) * _KB)]
                o_ref[0, :, b * _KB:(b + 1) * _KB] = (
                    o_ref[0, :, b * _KB:(b + 1) * _KB]
                    + jnp.concatenate(cols, axis=1))
            hit = jnp.logical_and(car[...] < (b + 1) * _KB, newcar > b * _KB)
            pl.when(jnp.any(hit))(band)
        car[...] = newcar

    cx = c_ref[0, :, 0:1]
    cy = c_ref[0, :, 1:2]
    cz = c_ref[0, :, 2:3]
    xj = x_ref[0, 0:1, :]
    yj = x_ref[0, 1:2, :]
    zj = x_ref[0, 2:3, :]
    d2 = (cx - xj) ** 2 + (cy - yj) ** 2 + (cz - zj) ** 2
    for r2, ns, o_ref, car in ((RAD2[0], NS[0], o1_ref, car1),
                               (RAD2[1], NS[1], o2_ref, car2)):
        mask = d2 <= r2
        cnt = jnp.sum(mask, axis=1, keepdims=True, dtype=jnp.int32)
        newcar = car[...] + cnt
        act = jnp.any(jnp.logical_and(car[...] < ns, cnt > 0))
        pl.when(act)(
            lambda mask=mask, ns=ns, o_ref=o_ref, car=car, newcar=newcar:
            scale_work(d2, mask, ns, o_ref, car, newcar))
        pl.when(jnp.logical_not(act))(
            lambda car=car, newcar=newcar: car.__setitem__(..., newcar))

    @pl.when(j == _NJ - 1)
    def _():
        kk1 = jax.lax.broadcasted_iota(jnp.int32, (1, NS[0]), 1)
        kk2 = jax.lax.broadcasted_iota(jnp.int32, (1, NS[1]), 1)
        for o_ref, car, kk in ((o1_ref, car1, kk1), (o2_ref, car2, kk2)):
            v = o_ref[0]
            first = v[:, 0:1]
            o_ref[0] = jnp.where(kk < car[...], v, first)


def _select(xyz_t, cents):
    return pl.pallas_call(
        _select_body,
        grid=(B, NPT // _CB, _NJ),
        in_specs=[
            pl.BlockSpec((1, 3, _JC), lambda b, c, j: (b, 0, j)),
            pl.BlockSpec((1, _CB, 3), lambda b, c, j: (b, c, 0)),
        ],
        out_specs=[
            pl.BlockSpec((1, _CB, NS[0]), lambda b, c, j: (b, c, 0)),
            pl.BlockSpec((1, _CB, NS[1]), lambda b, c, j: (b, c, 0)),
        ],
        out_shape=[
            jax.ShapeDtypeStruct((B, NPT, NS[0]), jnp.int32),
            jax.ShapeDtypeStruct((B, NPT, NS[1]), jnp.int32),
        ],
        scratch_shapes=[pltpu.VMEM((_CB, 1), jnp.int32),
                        pltpu.VMEM((_CB, 1), jnp.int32)],
    )(xyz_t, cents)


def _mm_bias_body(x_ref, w_ref, b_ref, o_ref):
    o_ref[...] = jnp.dot(x_ref[...], w_ref[...],
                         preferred_element_type=jnp.float32) + b_ref[...]


def _mm_bias(x, w, b2d, blk):
    rows, kdim = x.shape
    odim = w.shape[1]
    return pl.pallas_call(
        _mm_bias_body,
        grid=(rows // blk,),
        in_specs=[
            pl.BlockSpec((blk, kdim), lambda i: (i, 0)),
            pl.BlockSpec((kdim, odim), lambda i: (0, 0)),
            pl.BlockSpec((1, odim), lambda i: (0, 0)),
        ],
        out_specs=pl.BlockSpec((blk, odim), lambda i: (i, 0)),
        out_shape=jax.ShapeDtypeStruct((rows, odim), jnp.float32),
    )(x, w, b2d)


def _sc_gather(table, gidx, num):
    mesh = plsc.VectorSubcoreMesh(core_axis_name="c", subcore_axis_name="s")

    @functools.partial(
        pl.kernel,
        out_type=jax.ShapeDtypeStruct((num, 128), jnp.float32),
        mesh=mesh,
    )
    def _k(x_hbm, i_hbm, o_hbm):
        def body(i_vmem, o_vmem):
            pltpu.sync_copy(x_hbm.at[i_vmem.at[0]], o_vmem)

        pltpu.emit_pipeline(
            body,
            grid=(num // 128,),
            in_specs=[pl.BlockSpec((1, 128), lambda i: (0, i))],
            out_specs=[pl.BlockSpec((128, 128), lambda i: (i, 0))],
            core_axis_name=("c", "s"),
            dimension_semantics=(pltpu.PARALLEL,),
        )(i_hbm, o_hbm)

    return _k(table, gidx)


_CB5 = 128


def _tail_body(ns, g_ref, q_ref, w1_ref, b1_ref, w2_ref, b2_ref, o_ref):
    qb = q_ref[...]
    d1 = qb.shape[1]
    qrep = jnp.concatenate([qb] * ns, axis=0)
    h1 = jnp.maximum(g_ref[:, 0:d1] - qrep, 0.0)
    h2 = jnp.maximum(
        jnp.dot(h1, w1_ref[...], preferred_element_type=jnp.float32)
        + b1_ref[...], 0.0)
    h3 = jnp.maximum(
        jnp.dot(h2, w2_ref[...], preferred_element_type=jnp.float32)
        + b2_ref[...], 0.0)
    m = h3[0:_CB5]
    for k in range(1, ns):
        m = jnp.maximum(m, h3[k * _CB5:(k + 1) * _CB5])
    o_ref[...] = m


def _tail(g, q, w1, b1_2d, w2, b2_2d, ns):
    d1 = w1.shape[0]
    dm = w1.shape[1]
    do = w2.shape[1]
    nrows = _CB5 * ns
    return pl.pallas_call(
        functools.partial(_tail_body, ns),
        grid=(B * NPT // _CB5,),
        in_specs=[
            pl.BlockSpec((nrows, 128), lambda i: (i, 0)),
            pl.BlockSpec((_CB5, d1), lambda i: (i, 0)),
            pl.BlockSpec((d1, dm), lambda i: (0, 0)),
            pl.BlockSpec((1, dm), lambda i: (0, 0)),
            pl.BlockSpec((dm, do), lambda i: (0, 0)),
            pl.BlockSpec((1, do), lambda i: (0, 0)),
        ],
        out_specs=pl.BlockSpec((_CB5, do), lambda i: (i, 0)),
        out_shape=jax.ShapeDtypeStruct((B * NPT, do), jnp.float32),
    )(g, q, w1, b1_2d, w2, b2_2d)


def kernel(pointcloud, w1_0, b1_0, w1_1, b1_1, w1_2, b1_2,
           w2_0, b2_0, w2_1, b2_1, w2_2, b2_2):
    pc_t = jnp.transpose(pointcloud, (0, 2, 1))
    xyz_t = pc_t[:, :3, :]

    xyz_sq = xyz_t.reshape(B, 3, 128, 128)
    cxs, cys, czs = _fps(xyz_sq[:, 0], xyz_sq[:, 1], xyz_sq[:, 2])
    cents = jnp.stack([cxs, cys, czs], axis=-1)

    idx1, idx2 = _select(xyz_t, cents)

    pc_flat = pointcloud.reshape(B * N, 6)
    c_flat = cents.reshape(B * NPT, 3)
    boff = (jnp.arange(B, dtype=jnp.int32) * N)[:, None, None]

    outs = []
    params = ((w1_0, b1_0, w1_1, b1_1, w1_2, b1_2, idx1, NS[0]),
              (w2_0, b2_0, w2_1, b2_1, w2_2, b2_2, idx2, NS[1]))
    for w0, b0, w1, b1, w2, b2, idx, ns in params:
        w0p = jnp.pad(w0, ((0, 0), (0, 128 - w0.shape[1])))
        b0p = jnp.pad(b0.reshape(1, -1), ((0, 0), (0, 128 - w0.shape[1])))
        p_tab = _mm_bias(pc_flat, w0p, b0p, 1024)
        q_tab = _mm_bias(c_flat, w0[:3], jnp.zeros((1, w0.shape[1]),
                                                   jnp.float32), B * NPT)
        num = B * NPT * ns
        gidx = ((idx + boff).reshape(B * NPT, ns)
                .reshape(B * NPT // _CB5, _CB5, ns)
                .transpose(0, 2, 1).reshape(1, num))
        g = _sc_gather(p_tab, gidx, num)
        o = _tail(g, q_tab, w1, b1.reshape(1, -1),
                  w2, b2.reshape(1, -1), ns)
        outs.append(o.reshape(B, NPT, -1))

    return jnp.concatenate(outs, axis=-1)


# speedup vs baseline: 1.1725x; 1.1725x over previous
import functools

import jax
import jax.numpy as jnp
from jax.experimental import pallas as pl
from jax.experimental.pallas import tpu as pltpu
from jax.experimental.pallas import tpu_sc as plsc

B = 4
N = 16384
NPT = 1024
RAD2 = (0.25, 1.0)
NS = (16, 32)


def _fps_body(x_ref, y_ref, z_ref, ox_ref, oy_ref, oz_ref, d_ref):
    x = x_ref[...]
    y = y_ref[...]
    z = z_ref[...]
    lin = (jax.lax.broadcasted_iota(jnp.int32, (1, 128, 128), 1) * 128
           + jax.lax.broadcasted_iota(jnp.int32, (1, 128, 128), 2))
    ii = jax.lax.broadcasted_iota(jnp.int32, (B, NPT), 1)
    d_ref[...] = jnp.full((B, 128, 128), 1e10, jnp.float32)

    xyzcat = jnp.concatenate([x, y, z], axis=0)

    def step(i, far):
        sel = lin == far
        selcat = jnp.concatenate([sel, sel, sel], axis=0)
        csum = jnp.sum(jnp.where(selcat, xyzcat, 0.0),
                       axis=(1, 2), keepdims=True)
        cx = csum[0:B]
        cy = csum[B:2 * B]
        cz = csum[2 * B:3 * B]
        hit = ii == i
        ox_ref[...] = jnp.where(hit, cx[:, :, 0], ox_ref[...])
        oy_ref[...] = jnp.where(hit, cy[:, :, 0], oy_ref[...])
        oz_ref[...] = jnp.where(hit, cz[:, :, 0], oz_ref[...])
        d = (x - cx) ** 2 + (y - cy) ** 2 + (z - cz) ** 2
        dn = jnp.minimum(d_ref[...], d)
        d_ref[...] = dn
        m = jnp.max(dn, axis=(1, 2), keepdims=True)
        return jnp.min(jnp.where(dn == m, lin, jnp.int32(N)),
                       axis=(1, 2), keepdims=True)

    jax.lax.fori_loop(0, NPT, step, jnp.zeros((B, 1, 1), jnp.int32))


def _fps(xs, ys, zs):
    out = jax.ShapeDtypeStruct((B, NPT), jnp.float32)
    return pl.pallas_call(
        _fps_body,
        out_shape=[out, out, out],
        scratch_shapes=[pltpu.VMEM((B, 128, 128), jnp.float32)],
    )(xs, ys, zs)


_CB = 256
_JC = 2048
_NJ = N // _JC
_KB = 8


def _cumsum_lanes(m):
    x = m
    k = 1
    while k < _JC:
        x = x + jnp.concatenate(
            [jnp.zeros((_CB, k), jnp.int32), x[:, :-k]], axis=1)
        k *= 2
    return x


def _select_body(x_ref, c_ref, o1_ref, o2_ref, car1, car2):
    j = pl.program_id(2)

    @pl.when(j == 0)
    def _():
        car1[...] = jnp.zeros((_CB, 1), jnp.int32)
        car2[...] = jnp.zeros((_CB, 1), jnp.int32)
        o1_ref[...] = jnp.zeros((1, _CB, NS[0]), jnp.int32)
        o2_ref[...] = jnp.zeros((1, _CB, NS[1]), jnp.int32)

    act1 = jnp.min(car1[...]) < NS[0]
    act2 = jnp.min(car2[...]) < NS[1]

    def scale_work(d2, r2, ns, o_ref, car):
        mask = d2 <= r2
        pos = car[...] + _cumsum_lanes(mask.astype(jnp.int32))
        cols = [jnp.sum(pos <= k, axis=1, keepdims=True, dtype=jnp.int32)
                for k in range(ns)]
        o_ref[0] = o_ref[0] + jnp.concatenate(cols, axis=1)
        car[...] = pos[:, _JC - 1:_JC]

    @pl.when(jnp.logical_or(act1, act2))
    def _():
        cx = c_ref[0, :, 0:1]
        cy = c_ref[0, :, 1:2]
        cz = c_ref[0, :, 2:3]
        xj = x_ref[0, 0:1, :]
        yj = x_ref[0, 1:2, :]
        zj = x_ref[0, 2:3, :]
        d2 = (cx - xj) ** 2 + (cy - yj) ** 2 + (cz - zj) ** 2
        pl.when(act1)(
            lambda: scale_work(d2, RAD2[0], NS[0], o1_ref, car1))
        pl.when(act2)(
            lambda: scale_work(d2, RAD2[1], NS[1], o2_ref, car2))

    @pl.when(j == _NJ - 1)
    def _():
        for o_ref in (o1_ref, o2_ref):
            v = o_ref[0]
            first = v[:, 0:1]
            o_ref[0] = jnp.where(v == N, first, v)


def _select(xyz_t, cents):
    return pl.pallas_call(
        _select_body,
        grid=(B, NPT // _CB, _NJ),
        in_specs=[
            pl.BlockSpec((1, 3, _JC), lambda b, c, j: (b, 0, j)),
            pl.BlockSpec((1, _CB, 3), lambda b, c, j: (b, c, 0)),
        ],
        out_specs=[
            pl.BlockSpec((1, _CB, NS[0]), lambda b, c, j: (b, c, 0)),
            pl.BlockSpec((1, _CB, NS[1]), lambda b, c, j: (b, c, 0)),
        ],
        out_shape=[
            jax.ShapeDtypeStruct((B, NPT, NS[0]), jnp.int32),
            jax.ShapeDtypeStruct((B, NPT, NS[1]), jnp.int32),
        ],
        scratch_shapes=[pltpu.VMEM((_CB, 1), jnp.int32),
                        pltpu.VMEM((_CB, 1), jnp.int32)],
    )(xyz_t, cents)


def _mm_bias_body(x_ref, w_ref, b_ref, o_ref):
    o_ref[...] = jnp.dot(x_ref[...], w_ref[...],
                         preferred_element_type=jnp.float32) + b_ref[...]


def _mm_bias(x, w, b2d, blk):
    rows, kdim = x.shape
    odim = w.shape[1]
    return pl.pallas_call(
        _mm_bias_body,
        grid=(rows // blk,),
        in_specs=[
            pl.BlockSpec((blk, kdim), lambda i: (i, 0)),
            pl.BlockSpec((kdim, odim), lambda i: (0, 0)),
            pl.BlockSpec((1, odim), lambda i: (0, 0)),
        ],
        out_specs=pl.BlockSpec((blk, odim), lambda i: (i, 0)),
        out_shape=jax.ShapeDtypeStruct((rows, odim), jnp.float32),
    )(x, w, b2d)


def _sc_gather(table, gidx, num):
    mesh = plsc.VectorSubcoreMesh(core_axis_name="c", subcore_axis_name="s")

    @functools.partial(
        pl.kernel,
        out_type=jax.ShapeDtypeStruct((num, 128), jnp.float32),
        mesh=mesh,
    )
    def _k(x_hbm, i_hbm, o_hbm):
        def body(i_vmem, o_vmem):
            pltpu.sync_copy(x_hbm.at[i_vmem.at[0]], o_vmem)

        pltpu.emit_pipeline(
            body,
            grid=(num // 128,),
            in_specs=[pl.BlockSpec((1, 128), lambda i: (0, i))],
            out_specs=[pl.BlockSpec((128, 128), lambda i: (i, 0))],
            core_axis_name=("c", "s"),
            dimension_semantics=(pltpu.PARALLEL,),
        )(i_hbm, o_hbm)

    return _k(table, gidx)


_CB5 = 128


def _tail_body(ns, g_ref, q_ref, w1_ref, b1_ref, w2_ref, b2_ref, o_ref):
    qb = q_ref[...]
    d1 = qb.shape[1]
    qrep = jnp.concatenate([qb] * ns, axis=0)
    h1 = jnp.maximum(g_ref[:, 0:d1] - qrep, 0.0)
    h2 = jnp.maximum(
        jnp.dot(h1, w1_ref[...], preferred_element_type=jnp.float32)
        + b1_ref[...], 0.0)
    h3 = jnp.maximum(
        jnp.dot(h2, w2_ref[...], preferred_element_type=jnp.float32)
        + b2_ref[...], 0.0)
    m = h3[0:_CB5]
    for k in range(1, ns):
        m = jnp.maximum(m, h3[k * _CB5:(k + 1) * _CB5])
    o_ref[...] = m


def _tail(g, q, w1, b1_2d, w2, b2_2d, ns):
    d1 = w1.shape[0]
    dm = w1.shape[1]
    do = w2.shape[1]
    nrows = _CB5 * ns
    return pl.pallas_call(
        functools.partial(_tail_body, ns),
        grid=(B * NPT // _CB5,),
        in_specs=[
            pl.BlockSpec((nrows, 128), lambda i: (i, 0)),
            pl.BlockSpec((_CB5, d1), lambda i: (i, 0)),
            pl.BlockSpec((d1, dm), lambda i: (0, 0)),
            pl.BlockSpec((1, dm), lambda i: (0, 0)),
            pl.BlockSpec((dm, do), lambda i: (0, 0)),
            pl.BlockSpec((1, do), lambda i: (0, 0)),
        ],
        out_specs=pl.BlockSpec((_CB5, do), lambda i: (i, 0)),
        out_shape=jax.ShapeDtypeStruct((B * NPT, do), jnp.float32),
    )(g, q, w1, b1_2d, w2, b2_2d)


def kernel(pointcloud, w1_0, b1_0, w1_1, b1_1, w1_2, b1_2,
           w2_0, b2_0, w2_1, b2_1, w2_2, b2_2):
    pc_t = jnp.transpose(pointcloud, (0, 2, 1))
    xyz_t = pc_t[:, :3, :]

    xyz_sq = xyz_t.reshape(B, 3, 128, 128)
    cxs, cys, czs = _fps(xyz_sq[:, 0], xyz_sq[:, 1], xyz_sq[:, 2])
    cents = jnp.stack([cxs, cys, czs], axis=-1)

    idx1, idx2 = _select(xyz_t, cents)

    pc_flat = pointcloud.reshape(B * N, 6)
    c_flat = cents.reshape(B * NPT, 3)
    boff = (jnp.arange(B, dtype=jnp.int32) * N)[:, None, None]

    outs = []
    params = ((w1_0, b1_0, w1_1, b1_1, w1_2, b1_2, idx1, NS[0]),
              (w2_0, b2_0, w2_1, b2_1, w2_2, b2_2, idx2, NS[1]))
    for w0, b0, w1, b1, w2, b2, idx, ns in params:
        w0p = jnp.pad(w0, ((0, 0), (0, 128 - w0.shape[1])))
        b0p = jnp.pad(b0.reshape(1, -1), ((0, 0), (0, 128 - w0.shape[1])))
        p_tab = _mm_bias(pc_flat, w0p, b0p, 1024)
        q_tab = _mm_bias(c_flat, w0[:3], jnp.zeros((1, w0.shape[1]),
                                                   jnp.float32), B * NPT)
        num = B * NPT * ns
        gidx = ((idx + boff).reshape(B * NPT, ns)
                .reshape(B * NPT // _CB5, _CB5, ns)
                .transpose(0, 2, 1).reshape(1, num))
        g = _sc_gather(p_tab, gidx, num)
        o = _tail(g, q_tab, w1, b1.reshape(1, -1),
                  w2, b2.reshape(1, -1), ns)
        outs.append(o.reshape(B, NPT, -1))

    return jnp.concatenate(outs, axis=-1)
